# baseline (speedup 1.0000x reference)
import jax
import jax.numpy as jnp
from jax import lax
from jax.experimental import pallas as pl
from jax.experimental.pallas import tpu as pltpu
from jax.experimental.pallas import tpu_sc as plsc

B = 4096
S = 200
D = 32
C = 20
V = 1000000
NC = 2
NS = 16
NW = NC * NS
ROWS_PER_W = B // NW
CH = S // 2

NU = V // 128
TAIL = V - NU * 128
U_PER_W = NU // NW
U_EXTRA = NU % NW

def _i16():
    return jnp.arange(16, dtype=jnp.int32)


N_ITERS = U_PER_W + 2


def _format_body(tT_hbm, tail_hbm, out_hbm, inb, outb, tailb, sems):
    wid = lax.axis_index("subcore") * NC + lax.axis_index("core")
    u0 = wid * U_PER_W + jnp.minimum(wid, U_EXTRA)
    I16 = _i16()

    def unit(i):
        return jnp.minimum(u0 + i, NU - 1)

    def fire(u, b):
        pltpu.async_copy(
            tT_hbm.at[pl.ds(0, D), pl.ds(128 * u, 128)],
            inb.at[b], sems.at[b])

    def wait_in(b):
        pltpu.make_async_copy(
            tT_hbm.at[pl.ds(0, D), pl.ds(0, 128)],
            inb.at[b], sems.at[b]).wait()

    def wait_out(b):
        pltpu.make_async_copy(
            outb.at[b], out_hbm.at[pl.ds(0, 32)], sems.at[2 + b]).wait()

    fire(unit(0), 0)

    @pl.loop(0, N_ITERS, step=2)
    def _(i0):
        for db in range(2):
            i = i0 + db
            u = unit(i)

            @pl.when(i + 1 < N_ITERS)
            def _(i=i, db=db):
                fire(unit(i + 1), 1 - db)

            wait_in(db)

            @pl.when(i >= 2)
            def _(db=db):
                wait_out(db)

            dlo = I16
            dhi = I16 + 16

            @pl.loop(0, 128, step=8)
            def _(r0, db=db, dlo=dlo, dhi=dhi):
                for dr in range(8):
                    r = r0 + dr
                    rb = jnp.full((16,), 0, jnp.int32) + r
                    maj = r >> 2
                    mn = (r & 3) << 5
                    lo = plsc.load_gather(inb.at[db], [dlo, rb])
                    outb[db, maj, pl.ds(mn, 16)] = lo
                    hi = plsc.load_gather(inb.at[db], [dhi, rb])
                    outb[db, maj, pl.ds(mn + 16, 16)] = hi

            pltpu.async_copy(
                outb.at[db], out_hbm.at[pl.ds(32 * u, 32)], sems.at[2 + db])

    wait_out(0)
    wait_out(1)

    @pl.when(wid == 0)
    def _():
        pltpu.sync_copy(tail_hbm, tailb)
        pltpu.sync_copy(tailb, out_hbm.at[pl.ds(32 * NU, 16)])


NBUF = 4


def _pool_body(x_hbm, table_hbm, out_hbm, idx_v, rows_v, pooled_v, sems):
    wid = lax.axis_index("subcore") * NC + lax.axis_index("core")
    base = wid * ROWS_PER_W
    pltpu.sync_copy(x_hbm.at[pl.ds(base * 2, ROWS_PER_W * 2)], idx_v)

    def fire(r, b):
        pltpu.async_copy(
            table_hbm.at[idx_v.at[2 * r]],
            rows_v.at[b, pl.ds(0, CH)], sems.at[b])
        pltpu.async_copy(
            table_hbm.at[idx_v.at[2 * r + 1]],
            rows_v.at[b, pl.ds(CH, CH)], sems.at[b])

    def drain(b):
        pltpu.make_async_copy(
            table_hbm.at[pl.ds(0, S)], rows_v.at[b], sems.at[b]).wait()

    for b in range(NBUF):
        fire(b, b)

    @pl.loop(0, ROWS_PER_W, step=NBUF)
    def _(r0):
        for b in range(NBUF):
            drain(b)

            def red(j, acc):
                a0, a1, c0, c1 = acc
                j4 = 4 * j
                a0 = a0 + rows_v[b, j4, pl.ds(0, 16)]
                a1 = a1 + rows_v[b, j4, pl.ds(16, 16)]
                c0 = c0 + rows_v[b, j4 + 1, pl.ds(0, 16)]
                c1 = c1 + rows_v[b, j4 + 1, pl.ds(16, 16)]
                a0 = a0 + rows_v[b, j4 + 2, pl.ds(0, 16)]
                a1 = a1 + rows_v[b, j4 + 2, pl.ds(16, 16)]
                c0 = c0 + rows_v[b, j4 + 3, pl.ds(0, 16)]
                c1 = c1 + rows_v[b, j4 + 3, pl.ds(16, 16)]
                return (a0, a1, c0, c1)

            z = jnp.zeros((16,), jnp.float32)
            a0, a1, c0, c1 = lax.fori_loop(0, S // 4, red, (z, z, z, z))

            @pl.when(r0 < ROWS_PER_W - NBUF)
            def _():
                fire(r0 + NBUF + b, b)

            pooled_v[r0 + b, pl.ds(0, 16)] = a0 + c0
            pooled_v[r0 + b, pl.ds(16, 16)] = a1 + c1

    pltpu.sync_copy(pooled_v, out_hbm.at[pl.ds(base, ROWS_PER_W)])


def _linear_body(p_ref, w_ref, b_ref, o_ref):
    o_ref[...] = (
        jnp.dot(p_ref[...], w_ref[...], preferred_element_type=jnp.float32)
        + b_ref[...])


def kernel(x, table, W, b):
    x2 = x.astype(jnp.int32).reshape(B * 2, CH)
    mesh = plsc.VectorSubcoreMesh(
        core_axis_name="core", subcore_axis_name="subcore")

    tT = table.T
    tail = table[NU * 128:].reshape(16, 128)

    fmt = pl.kernel(
        _format_body,
        out_type=jax.ShapeDtypeStruct((V // 4, 128), jnp.float32),
        mesh=mesh,
        scratch_types=[
            pltpu.VMEM((2, 32, 128), jnp.float32),
            pltpu.VMEM((2, 32, 128), jnp.float32),
            pltpu.VMEM((16, 128), jnp.float32),
            pltpu.SemaphoreType.DMA((4,)),
        ],
        compiler_params=pltpu.CompilerParams(
            use_tc_tiling_on_sc=True, needs_layout_passes=False),
    )
    table_rm = fmt(tT, tail).reshape(V, D)

    pool = pl.kernel(
        _pool_body,
        out_type=jax.ShapeDtypeStruct((B, D), jnp.float32),
        mesh=mesh,
        scratch_types=[
            pltpu.VMEM((2 * ROWS_PER_W, CH), jnp.int32),
            pltpu.VMEM((NBUF, S, D), jnp.float32),
            pltpu.VMEM((ROWS_PER_W, D), jnp.float32),
            pltpu.SemaphoreType.DMA((NBUF,)),
        ],
        compiler_params=pltpu.CompilerParams(use_tc_tiling_on_sc=False),
    )
    pooled = pool(x2, table_rm)

    logits = pl.pallas_call(
        _linear_body,
        out_shape=jax.ShapeDtypeStruct((B, C), jnp.float32),
    )(pooled, W, b.reshape(1, C))
    return logits


# speedup vs baseline: 3.2597x; 3.2597x over previous
import jax
import jax.numpy as jnp
from jax import lax
from jax.experimental import pallas as pl
from jax.experimental.pallas import tpu as pltpu
from jax.experimental.pallas import tpu_sc as plsc

B = 4096
S = 200
D = 32
C = 20
V = 1000000
NC = 2
NS = 16
NW = NC * NS
ROWS_PER_W = B // NW
CH = S // 2

NU = V // 128
TAIL = V - NU * 128
U_PER_W = NU // NW
U_EXTRA = NU % NW

def _i16():
    return jnp.arange(16, dtype=jnp.int32)


N_ITERS = U_PER_W + 2


def _format_body(tT_hbm, tail_hbm, out_hbm, inb, outb, tailb, sems):
    wid = lax.axis_index("subcore") * NC + lax.axis_index("core")
    u0 = wid * U_PER_W + jnp.minimum(wid, U_EXTRA)
    I16 = _i16()

    def unit(i):
        return jnp.minimum(u0 + i, NU - 1)

    def fire(u, b):
        pltpu.async_copy(
            tT_hbm.at[pl.ds(0, D), pl.ds(128 * u, 128)],
            inb.at[b], sems.at[b])

    def wait_in(b):
        pltpu.make_async_copy(
            tT_hbm.at[pl.ds(0, D), pl.ds(0, 128)],
            inb.at[b], sems.at[b]).wait()

    def wait_out(b):
        pltpu.make_async_copy(
            outb.at[b], out_hbm.at[pl.ds(0, 32)], sems.at[2 + b]).wait()

    fire(unit(0), 0)

    @pl.loop(0, N_ITERS, step=2)
    def _(i0):
        for db in range(2):
            i = i0 + db
            u = unit(i)

            @pl.when(i + 1 < N_ITERS)
            def _(i=i, db=db):
                fire(unit(i + 1), 1 - db)

            wait_in(db)

            @pl.when(i >= 2)
            def _(db=db):
                wait_out(db)

            for d0 in (0, 16):
                dvec = I16 + d0

                @pl.loop(0, 128, step=16)
                def _(r0, db=db, d0=d0, dvec=dvec):
                    r0q = r0 >> 2
                    srcs = []
                    idxs = []
                    for k in range(16):
                        perm = (I16 + k) & 15
                        gdst = (perm << 5) + dvec
                        srcs.append(
                            plsc.load_gather(inb.at[db], [dvec, perm + r0]))
                        idxs.append(((gdst >> 7) + r0q, gdst & 127))
                    for k in range(16):
                        mj, mn = idxs[k]
                        plsc.store_scatter(outb.at[db], [mj, mn], srcs[k])

            pltpu.async_copy(
                outb.at[db], out_hbm.at[pl.ds(32 * u, 32)], sems.at[2 + db])

    wait_out(0)
    wait_out(1)

    @pl.when(wid == 0)
    def _():
        pltpu.sync_copy(tail_hbm, tailb)
        pltpu.sync_copy(tailb, out_hbm.at[pl.ds(32 * NU, 16)])


NBUF = 4


def _pool_body(x_hbm, table_hbm, out_hbm, idx_v, rows_v, pooled_v, sems):
    wid = lax.axis_index("subcore") * NC + lax.axis_index("core")
    base = wid * ROWS_PER_W
    pltpu.sync_copy(x_hbm.at[pl.ds(base * 2, ROWS_PER_W * 2)], idx_v)

    def fire(r, b):
        pltpu.async_copy(
            table_hbm.at[idx_v.at[2 * r]],
            rows_v.at[b, pl.ds(0, CH)], sems.at[b])
        pltpu.async_copy(
            table_hbm.at[idx_v.at[2 * r + 1]],
            rows_v.at[b, pl.ds(CH, CH)], sems.at[b])

    def drain(b):
        pltpu.make_async_copy(
            table_hbm.at[pl.ds(0, S)], rows_v.at[b], sems.at[b]).wait()

    for b in range(NBUF):
        fire(b, b)

    @pl.loop(0, ROWS_PER_W, step=NBUF)
    def _(r0):
        for b in range(NBUF):
            drain(b)

            def red(j, acc):
                a0, a1, c0, c1 = acc
                j4 = 4 * j
                a0 = a0 + rows_v[b, j4, pl.ds(0, 16)]
                a1 = a1 + rows_v[b, j4, pl.ds(16, 16)]
                c0 = c0 + rows_v[b, j4 + 1, pl.ds(0, 16)]
                c1 = c1 + rows_v[b, j4 + 1, pl.ds(16, 16)]
                a0 = a0 + rows_v[b, j4 + 2, pl.ds(0, 16)]
                a1 = a1 + rows_v[b, j4 + 2, pl.ds(16, 16)]
                c0 = c0 + rows_v[b, j4 + 3, pl.ds(0, 16)]
                c1 = c1 + rows_v[b, j4 + 3, pl.ds(16, 16)]
                return (a0, a1, c0, c1)

            z = jnp.zeros((16,), jnp.float32)
            a0, a1, c0, c1 = lax.fori_loop(0, S // 4, red, (z, z, z, z))

            @pl.when(r0 < ROWS_PER_W - NBUF)
            def _():
                fire(r0 + NBUF + b, b)

            pooled_v[r0 + b, pl.ds(0, 16)] = a0 + c0
            pooled_v[r0 + b, pl.ds(16, 16)] = a1 + c1

    pltpu.sync_copy(pooled_v, out_hbm.at[pl.ds(base, ROWS_PER_W)])


def _linear_body(p_ref, w_ref, b_ref, o_ref):
    o_ref[...] = (
        jnp.dot(p_ref[...], w_ref[...], preferred_element_type=jnp.float32)
        + b_ref[...])


def kernel(x, table, W, b):
    x2 = x.astype(jnp.int32).reshape(B * 2, CH)
    mesh = plsc.VectorSubcoreMesh(
        core_axis_name="core", subcore_axis_name="subcore")

    tT = table.T
    tail = table[NU * 128:].reshape(16, 128)

    fmt = pl.kernel(
        _format_body,
        out_type=jax.ShapeDtypeStruct((V // 4, 128), jnp.float32),
        mesh=mesh,
        scratch_types=[
            pltpu.VMEM((2, 32, 128), jnp.float32),
            pltpu.VMEM((2, 32, 128), jnp.float32),
            pltpu.VMEM((16, 128), jnp.float32),
            pltpu.SemaphoreType.DMA((4,)),
        ],
        compiler_params=pltpu.CompilerParams(
            use_tc_tiling_on_sc=True, needs_layout_passes=False),
    )
    table_rm = fmt(tT, tail).reshape(V, D)

    pool = pl.kernel(
        _pool_body,
        out_type=jax.ShapeDtypeStruct((B, D), jnp.float32),
        mesh=mesh,
        scratch_types=[
            pltpu.VMEM((2 * ROWS_PER_W, CH), jnp.int32),
            pltpu.VMEM((NBUF, S, D), jnp.float32),
            pltpu.VMEM((ROWS_PER_W, D), jnp.float32),
            pltpu.SemaphoreType.DMA((NBUF,)),
        ],
        compiler_params=pltpu.CompilerParams(use_tc_tiling_on_sc=False),
    )
    pooled = pool(x2, table_rm)

    logits = pl.pallas_call(
        _linear_body,
        out_shape=jax.ShapeDtypeStruct((B, C), jnp.float32),
    )(pooled, W, b.reshape(1, C))
    return logits


# speedup vs baseline: 3.3188x; 1.0181x over previous
import jax
import jax.numpy as jnp
from jax import lax
from jax.experimental import pallas as pl
from jax.experimental.pallas import tpu as pltpu
from jax.experimental.pallas import tpu_sc as plsc

B = 4096
S = 200
D = 32
C = 20
V = 1000000
NC = 2
NS = 16
NW = NC * NS
ROWS_PER_W = B // NW
CH = S // 2

NU = V // 128
TAIL = V - NU * 128
U_PER_W = NU // NW
U_EXTRA = NU % NW

def _i16():
    return jnp.arange(16, dtype=jnp.int32)


N_ITERS = U_PER_W + 2


def _format_body(tT_hbm, tail_hbm, out_hbm, inb, outb, tailb, sems):
    wid = lax.axis_index("subcore") * NC + lax.axis_index("core")
    u0 = wid * U_PER_W + jnp.minimum(wid, U_EXTRA)
    I16 = _i16()

    def unit(i):
        return jnp.minimum(u0 + i, NU - 1)

    def fire(u, b):
        pltpu.async_copy(
            tT_hbm.at[pl.ds(0, D), pl.ds(128 * u, 128)],
            inb.at[b], sems.at[b])

    def wait_in(b):
        pltpu.make_async_copy(
            tT_hbm.at[pl.ds(0, D), pl.ds(0, 128)],
            inb.at[b], sems.at[b]).wait()

    def wait_out(b):
        pltpu.make_async_copy(
            outb.at[b], out_hbm.at[pl.ds(0, 32)], sems.at[2 + b]).wait()

    fire(unit(0), 0)

    @pl.loop(0, N_ITERS, step=2)
    def _(i0):
        for db in range(2):
            i = i0 + db
            u = unit(i)

            @pl.when(i + 1 < N_ITERS)
            def _(i=i, db=db):
                fire(unit(i + 1), 1 - db)

            wait_in(db)

            @pl.when(i >= 2)
            def _(db=db):
                wait_out(db)

            for d0 in (0, 16):
                dvec = I16 + d0

                @pl.loop(0, 128, step=16)
                def _(r0, db=db, d0=d0, dvec=dvec):
                    r0q = r0 >> 2
                    for g in range(0, 16, 4):
                        srcs = []
                        idxs = []
                        for k in range(g, g + 4):
                            perm = (I16 + k) & 15
                            gdst = (perm << 5) + dvec
                            srcs.append(plsc.load_gather(
                                inb.at[db], [dvec, perm + r0]))
                            idxs.append(((gdst >> 7) + r0q, gdst & 127))
                        for kk in range(4):
                            mj, mn = idxs[kk]
                            plsc.store_scatter(
                                outb.at[db], [mj, mn], srcs[kk])

            pltpu.async_copy(
                outb.at[db], out_hbm.at[pl.ds(32 * u, 32)], sems.at[2 + db])

    wait_out(0)
    wait_out(1)

    @pl.when(wid == 0)
    def _():
        pltpu.sync_copy(tail_hbm, tailb)
        pltpu.sync_copy(tailb, out_hbm.at[pl.ds(32 * NU, 16)])


NBUF = 4


def _pool_body(x_hbm, table_hbm, out_hbm, idx_v, rows_v, pooled_v, sems):
    wid = lax.axis_index("subcore") * NC + lax.axis_index("core")
    base = wid * ROWS_PER_W
    pltpu.sync_copy(x_hbm.at[pl.ds(base * 2, ROWS_PER_W * 2)], idx_v)

    def fire(r, b):
        pltpu.async_copy(
            table_hbm.at[idx_v.at[2 * r]],
            rows_v.at[b, pl.ds(0, CH)], sems.at[b])
        pltpu.async_copy(
            table_hbm.at[idx_v.at[2 * r + 1]],
            rows_v.at[b, pl.ds(CH, CH)], sems.at[b])

    def drain(b):
        pltpu.make_async_copy(
            table_hbm.at[pl.ds(0, S)], rows_v.at[b], sems.at[b]).wait()

    for b in range(NBUF):
        fire(b, b)

    @pl.loop(0, ROWS_PER_W, step=NBUF)
    def _(r0):
        for b in range(NBUF):
            drain(b)

            def red(j, acc):
                a0, a1, c0, c1 = acc
                j4 = 4 * j
                a0 = a0 + rows_v[b, j4, pl.ds(0, 16)]
                a1 = a1 + rows_v[b, j4, pl.ds(16, 16)]
                c0 = c0 + rows_v[b, j4 + 1, pl.ds(0, 16)]
                c1 = c1 + rows_v[b, j4 + 1, pl.ds(16, 16)]
                a0 = a0 + rows_v[b, j4 + 2, pl.ds(0, 16)]
                a1 = a1 + rows_v[b, j4 + 2, pl.ds(16, 16)]
                c0 = c0 + rows_v[b, j4 + 3, pl.ds(0, 16)]
                c1 = c1 + rows_v[b, j4 + 3, pl.ds(16, 16)]
                return (a0, a1, c0, c1)

            z = jnp.zeros((16,), jnp.float32)
            a0, a1, c0, c1 = lax.fori_loop(0, S // 4, red, (z, z, z, z))

            @pl.when(r0 < ROWS_PER_W - NBUF)
            def _():
                fire(r0 + NBUF + b, b)

            pooled_v[r0 + b, pl.ds(0, 16)] = a0 + c0
            pooled_v[r0 + b, pl.ds(16, 16)] = a1 + c1

    pltpu.sync_copy(pooled_v, out_hbm.at[pl.ds(base, ROWS_PER_W)])


def _linear_body(p_ref, w_ref, b_ref, o_ref):
    o_ref[...] = (
        jnp.dot(p_ref[...], w_ref[...], preferred_element_type=jnp.float32)
        + b_ref[...])


def kernel(x, table, W, b):
    x2 = x.astype(jnp.int32).reshape(B * 2, CH)
    mesh = plsc.VectorSubcoreMesh(
        core_axis_name="core", subcore_axis_name="subcore")

    tT = table.T
    tail = table[NU * 128:].reshape(16, 128)

    fmt = pl.kernel(
        _format_body,
        out_type=jax.ShapeDtypeStruct((V // 4, 128), jnp.float32),
        mesh=mesh,
        scratch_types=[
            pltpu.VMEM((2, 32, 128), jnp.float32),
            pltpu.VMEM((2, 32, 128), jnp.float32),
            pltpu.VMEM((16, 128), jnp.float32),
            pltpu.SemaphoreType.DMA((4,)),
        ],
        compiler_params=pltpu.CompilerParams(
            use_tc_tiling_on_sc=True, needs_layout_passes=False),
    )
    table_rm = fmt(tT, tail).reshape(V, D)

    pool = pl.kernel(
        _pool_body,
        out_type=jax.ShapeDtypeStruct((B, D), jnp.float32),
        mesh=mesh,
        scratch_types=[
            pltpu.VMEM((2 * ROWS_PER_W, CH), jnp.int32),
            pltpu.VMEM((NBUF, S, D), jnp.float32),
            pltpu.VMEM((ROWS_PER_W, D), jnp.float32),
            pltpu.SemaphoreType.DMA((NBUF,)),
        ],
        compiler_params=pltpu.CompilerParams(use_tc_tiling_on_sc=False),
    )
    pooled = pool(x2, table_rm)

    logits = pl.pallas_call(
        _linear_body,
        out_shape=jax.ShapeDtypeStruct((B, C), jnp.float32),
    )(pooled, W, b.reshape(1, C))
    return logits


# speedup vs baseline: 3.4231x; 1.0314x over previous
import jax
import jax.numpy as jnp
from jax import lax
from jax.experimental import pallas as pl
from jax.experimental.pallas import tpu as pltpu
from jax.experimental.pallas import tpu_sc as plsc

B = 4096
S = 200
D = 32
C = 20
V = 1000000
NC = 2
NS = 16
NW = NC * NS
ROWS_PER_W = B // NW
CH = S // 2

NU = V // 128
TAIL = V - NU * 128
U_PER_W = NU // NW
U_EXTRA = NU % NW

def _i16():
    return jnp.arange(16, dtype=jnp.int32)


N_ITERS = U_PER_W + 2


def _format_body(tT_hbm, tail_hbm, out_hbm, inb, outb, tailb, sems):
    wid = lax.axis_index("subcore") * NC + lax.axis_index("core")
    u0 = wid * U_PER_W + jnp.minimum(wid, U_EXTRA)
    I16 = _i16()

    def unit(i):
        return jnp.minimum(u0 + i, NU - 1)

    def fire(u, b):
        pltpu.async_copy(
            tT_hbm.at[pl.ds(0, D), pl.ds(128 * u, 128)],
            inb.at[b], sems.at[b])

    def wait_in(b):
        pltpu.make_async_copy(
            tT_hbm.at[pl.ds(0, D), pl.ds(0, 128)],
            inb.at[b], sems.at[b]).wait()

    def wait_out(b):
        pltpu.make_async_copy(
            outb.at[b], out_hbm.at[pl.ds(0, 32)], sems.at[2 + b]).wait()

    fire(unit(0), 0)

    @pl.loop(0, N_ITERS, step=2)
    def _(i0):
        for db in range(2):
            i = i0 + db
            u = unit(i)

            @pl.when(i + 1 < N_ITERS)
            def _(i=i, db=db):
                fire(unit(i + 1), 1 - db)

            wait_in(db)

            @pl.when(i >= 2)
            def _(db=db):
                wait_out(db)

            for d0 in (0, 16):
                dvec = I16 + d0

                @pl.loop(0, 128, step=16)
                def _(r0, db=db, d0=d0, dvec=dvec):
                    r0q = r0 >> 2
                    for g in range(0, 16, 4):
                        srcs = []
                        idxs = []
                        for k in range(g, g + 4):
                            perm = (I16 + k) & 15
                            gdst = (perm << 5) + dvec
                            srcs.append(plsc.load_gather(
                                inb.at[db], [dvec, perm + r0]))
                            idxs.append(((gdst >> 7) + r0q, gdst & 127))
                        for kk in range(4):
                            mj, mn = idxs[kk]
                            plsc.store_scatter(
                                outb.at[db], [mj, mn], srcs[kk])

            pltpu.async_copy(
                outb.at[db], out_hbm.at[pl.ds(32 * u, 32)], sems.at[2 + db])

    wait_out(0)
    wait_out(1)

    @pl.when(wid == 0)
    def _():
        pltpu.sync_copy(tail_hbm, tailb)
        pltpu.sync_copy(tailb, out_hbm.at[pl.ds(32 * NU, 16)])


NBUF = 8


def _pool_body(x_hbm, table_hbm, out_hbm, idx_v, rows_v, pooled_v, sems):
    wid = lax.axis_index("subcore") * NC + lax.axis_index("core")
    base = wid * ROWS_PER_W
    pltpu.sync_copy(x_hbm.at[pl.ds(base * 2, ROWS_PER_W * 2)], idx_v)

    def fire(r, b):
        pltpu.async_copy(
            table_hbm.at[idx_v.at[2 * r]],
            rows_v.at[b, pl.ds(0, CH)], sems.at[b])
        pltpu.async_copy(
            table_hbm.at[idx_v.at[2 * r + 1]],
            rows_v.at[b, pl.ds(CH, CH)], sems.at[b])

    def drain(b):
        pltpu.make_async_copy(
            table_hbm.at[pl.ds(0, S)], rows_v.at[b], sems.at[b]).wait()

    for b in range(NBUF):
        fire(b, b)

    @pl.loop(0, ROWS_PER_W, step=NBUF)
    def _(r0):
        for b in range(NBUF):
            drain(b)

            def red(j, acc):
                a0, a1, c0, c1 = acc
                j4 = 4 * j
                a0 = a0 + rows_v[b, j4, pl.ds(0, 16)]
                a1 = a1 + rows_v[b, j4, pl.ds(16, 16)]
                c0 = c0 + rows_v[b, j4 + 1, pl.ds(0, 16)]
                c1 = c1 + rows_v[b, j4 + 1, pl.ds(16, 16)]
                a0 = a0 + rows_v[b, j4 + 2, pl.ds(0, 16)]
                a1 = a1 + rows_v[b, j4 + 2, pl.ds(16, 16)]
                c0 = c0 + rows_v[b, j4 + 3, pl.ds(0, 16)]
                c1 = c1 + rows_v[b, j4 + 3, pl.ds(16, 16)]
                return (a0, a1, c0, c1)

            z = jnp.zeros((16,), jnp.float32)
            a0, a1, c0, c1 = lax.fori_loop(0, S // 4, red, (z, z, z, z))

            @pl.when(r0 < ROWS_PER_W - NBUF)
            def _():
                fire(r0 + NBUF + b, b)

            pooled_v[r0 + b, pl.ds(0, 16)] = a0 + c0
            pooled_v[r0 + b, pl.ds(16, 16)] = a1 + c1

    pltpu.sync_copy(pooled_v, out_hbm.at[pl.ds(base, ROWS_PER_W)])


def _linear_body(p_ref, w_ref, b_ref, o_ref):
    o_ref[...] = (
        jnp.dot(p_ref[...], w_ref[...], preferred_element_type=jnp.float32)
        + b_ref[...])


def kernel(x, table, W, b):
    x2 = x.astype(jnp.int32).reshape(B * 2, CH)
    mesh = plsc.VectorSubcoreMesh(
        core_axis_name="core", subcore_axis_name="subcore")

    tT = table.T
    tail = table[NU * 128:].reshape(16, 128)

    fmt = pl.kernel(
        _format_body,
        out_type=jax.ShapeDtypeStruct((V // 4, 128), jnp.float32),
        mesh=mesh,
        scratch_types=[
            pltpu.VMEM((2, 32, 128), jnp.float32),
            pltpu.VMEM((2, 32, 128), jnp.float32),
            pltpu.VMEM((16, 128), jnp.float32),
            pltpu.SemaphoreType.DMA((4,)),
        ],
        compiler_params=pltpu.CompilerParams(
            use_tc_tiling_on_sc=True, needs_layout_passes=False),
    )
    table_rm = fmt(tT, tail).reshape(V, D)

    pool = pl.kernel(
        _pool_body,
        out_type=jax.ShapeDtypeStruct((B, D), jnp.float32),
        mesh=mesh,
        scratch_types=[
            pltpu.VMEM((2 * ROWS_PER_W, CH), jnp.int32),
            pltpu.VMEM((NBUF, S, D), jnp.float32),
            pltpu.VMEM((ROWS_PER_W, D), jnp.float32),
            pltpu.SemaphoreType.DMA((NBUF,)),
        ],
        compiler_params=pltpu.CompilerParams(use_tc_tiling_on_sc=False),
    )
    pooled = pool(x2, table_rm)

    logits = pl.pallas_call(
        _linear_body,
        out_shape=jax.ShapeDtypeStruct((B, C), jnp.float32),
    )(pooled, W, b.reshape(1, C))
    return logits
